# baseline (speedup 1.0000x reference)
import functools

import jax
import jax.numpy as jnp
from jax import lax
from jax.experimental import pallas as pl
from jax.experimental.pallas import tpu as pltpu
from jax.experimental.pallas import tpu_sc as plsc

VOCAB = 1000
HIDDEN = 256

NC = 2
NS = 16
NW = NC * NS

C = 128
NCH = 50
D = HIDDEN
HC = C // 2


def _proj_kernel(e_ref, w_ref, b_ref, out_ref):
    out_ref[...] = lax.dot_general(
        e_ref[...], w_ref[...],
        dimension_numbers=(((1,), (1,)), ((), ())),
        preferred_element_type=jnp.float32,
    ) + b_ref[...]


def _project_table(E, W, b2d):
    return pl.pallas_call(
        _proj_kernel,
        out_shape=jax.ShapeDtypeStruct((VOCAB, HIDDEN), jnp.float32),
    )(E, W, b2d)


def _gather_body(table_hbm, idx_hbm, out_hbm, shr, idx_v,
                 rows0, rows1, gsem0, gsem1, bsem0, bsem1, dsem0, dsem1):
    wid = lax.axis_index("s") * NC + lax.axis_index("c")
    sid = lax.axis_index("s")
    rows = (rows0, rows1)
    gsem = (gsem0, gsem1)
    bsem = (bsem0, bsem1)
    dsem = (dsem0, dsem1)
    spm = (shr.at[sid, 0], shr.at[sid, 1])
    row0 = wid * (NCH * C)
    pltpu.sync_copy(idx_hbm.at[wid], idx_v)

    g0 = pltpu.async_copy(table_hbm.at[idx_v.at[0]], rows[0], gsem0)
    g1 = pltpu.async_copy(table_hbm.at[idx_v.at[1]], rows[1], gsem1)
    del g0, g1

    @pl.loop(0, NCH, step=2)
    def _(ch):
        cur0 = ch
        pltpu.make_async_copy(table_hbm.at[idx_v.at[cur0]],
                              rows[0], gsem[0]).wait()
        pltpu.async_copy(rows[0], out_hbm.at[pl.ds(row0 + cur0 * C, C)],
                         bsem[0])

        cur1 = ch + 1
        pltpu.make_async_copy(table_hbm.at[idx_v.at[cur1]],
                              rows[1], gsem[1]).wait()
        for h in range(2):
            @pl.when(cur1 >= 3)
            def _():
                pltpu.make_async_copy(
                    spm[h],
                    out_hbm.at[pl.ds(row0 + (cur1 - 2) * C + h * HC, HC)],
                    dsem[h]).wait()

            half = rows[1].at[pl.ds(h * HC, HC)]
            pltpu.async_copy(half, spm[h], bsem[1])
            pltpu.make_async_copy(half, spm[h], bsem[1]).wait()
            pltpu.async_copy(
                spm[h],
                out_hbm.at[pl.ds(row0 + cur1 * C + h * HC, HC)],
                dsem[h])

        @pl.when(cur0 + 2 < NCH)
        def _():
            pltpu.make_async_copy(
                rows[0], out_hbm.at[pl.ds(row0 + cur0 * C, C)],
                bsem[0]).wait()
            pltpu.async_copy(table_hbm.at[idx_v.at[cur0 + 2]],
                             rows[0], gsem[0])

        @pl.when(cur1 + 2 < NCH)
        def _():
            pltpu.async_copy(table_hbm.at[idx_v.at[cur1 + 2]],
                             rows[1], gsem[1])

    pltpu.make_async_copy(
        rows[0], out_hbm.at[pl.ds(row0 + (NCH - 2) * C, C)], bsem[0]).wait()
    for h in range(2):
        pltpu.make_async_copy(
            spm[h],
            out_hbm.at[pl.ds(row0 + (NCH - 1) * C + h * HC, HC)],
            dsem[h]).wait()


@functools.cache
def _gather():
    return pl.kernel(
        _gather_body,
        out_type=jax.ShapeDtypeStruct((NW * NCH * C, D), jnp.float32),
        mesh=plsc.VectorSubcoreMesh(
            core_axis_name="c", subcore_axis_name="s",
            num_cores=NC, num_subcores=NS),
        compiler_params=pltpu.CompilerParams(use_tc_tiling_on_sc=True),
        scratch_types=[
            pltpu.VMEM_SHARED((NS, 2, HC, D), jnp.float32),
            pltpu.VMEM((NCH, C), jnp.int32),
            pltpu.VMEM((C, D), jnp.float32),
            pltpu.VMEM((C, D), jnp.float32),
            pltpu.SemaphoreType.DMA,
            pltpu.SemaphoreType.DMA,
            pltpu.SemaphoreType.DMA,
            pltpu.SemaphoreType.DMA,
            pltpu.SemaphoreType.DMA,
            pltpu.SemaphoreType.DMA,
        ],
    )


def kernel(input_ids, embed_table, W, b):
    B, S = input_ids.shape
    P = _project_table(embed_table, W, b.reshape(1, HIDDEN))
    idx = input_ids.T.reshape(NW, NCH, C).astype(jnp.int32)
    out = _gather()(P, idx)
    return out.reshape(S, B, HIDDEN).transpose(1, 0, 2)


# speedup vs baseline: 1.0353x; 1.0353x over previous
import functools

import jax
import jax.numpy as jnp
from jax import lax
from jax.experimental import pallas as pl
from jax.experimental.pallas import tpu as pltpu
from jax.experimental.pallas import tpu_sc as plsc

VOCAB = 1000
HIDDEN = 256

NC = 2
NS = 16
NW = NC * NS

C = 128
NCH = 50
D = HIDDEN
HC = C // 4


def _proj_kernel(e_ref, w_ref, b_ref, out_ref):
    out_ref[...] = lax.dot_general(
        e_ref[...], w_ref[...],
        dimension_numbers=(((1,), (1,)), ((), ())),
        preferred_element_type=jnp.float32,
    ) + b_ref[...]


def _project_table(E, W, b2d):
    return pl.pallas_call(
        _proj_kernel,
        out_shape=jax.ShapeDtypeStruct((VOCAB, HIDDEN), jnp.float32),
    )(E, W, b2d)


def _gather_body(table_hbm, idx_hbm, out_hbm, shr, idx_v,
                 rows0, rows1, rows2,
                 gsem0, gsem1, gsem2, bsem0, bsem1, dsem0, dsem1):
    wid = lax.axis_index("s") * NC + lax.axis_index("c")
    sid = lax.axis_index("s")
    rows = (rows0, rows1, rows2)
    gsem = (gsem0, gsem1, gsem2)
    bsem = (bsem0, bsem1)
    dsem = (dsem0, dsem1)
    spm = (shr.at[sid, 0], shr.at[sid, 1])
    row0 = wid * (NCH * C)
    pltpu.sync_copy(idx_hbm.at[wid], idx_v)

    for k in range(3):
        pltpu.async_copy(table_hbm.at[idx_v.at[k]], rows[k], gsem[k])

    def step(cur, b):
        pltpu.make_async_copy(table_hbm.at[idx_v.at[cur]],
                              rows[b], gsem[b]).wait()

        for h in range(4):
            @pl.when(cur * 4 + h >= 2)
            def _():
                pltpu.make_async_copy(
                    spm[h % 2],
                    out_hbm.at[pl.ds(row0 + cur * C + (h - 2) * HC, HC)],
                    dsem[h % 2]).wait()

            part = rows[b].at[pl.ds(h * HC, HC)]
            pltpu.async_copy(part, spm[h % 2], bsem[h % 2])
            pltpu.make_async_copy(part, spm[h % 2], bsem[h % 2]).wait()
            pltpu.async_copy(
                spm[h % 2],
                out_hbm.at[pl.ds(row0 + cur * C + h * HC, HC)],
                dsem[h % 2])

        @pl.when(cur + 3 < NCH)
        def _():
            pltpu.async_copy(table_hbm.at[idx_v.at[cur + 3]],
                             rows[b], gsem[b])

    @pl.loop(0, NCH - 2, step=3)
    def _(ch):
        for b in range(3):
            step(ch + b, b)

    step(NCH - 2, 0)
    step(NCH - 1, 1)

    for h in (2, 3):
        pltpu.make_async_copy(
            spm[h % 2],
            out_hbm.at[pl.ds(row0 + (NCH - 1) * C + h * HC, HC)],
            dsem[h % 2]).wait()


@functools.cache
def _gather():
    return pl.kernel(
        _gather_body,
        out_type=jax.ShapeDtypeStruct((NW * NCH * C, D), jnp.float32),
        mesh=plsc.VectorSubcoreMesh(
            core_axis_name="c", subcore_axis_name="s",
            num_cores=NC, num_subcores=NS),
        compiler_params=pltpu.CompilerParams(use_tc_tiling_on_sc=True),
        scratch_types=[
            pltpu.VMEM_SHARED((NS, 2, HC, D), jnp.float32),
            pltpu.VMEM((NCH, C), jnp.int32),
            pltpu.VMEM((C, D), jnp.float32),
            pltpu.VMEM((C, D), jnp.float32),
            pltpu.VMEM((C, D), jnp.float32),
            pltpu.SemaphoreType.DMA,
            pltpu.SemaphoreType.DMA,
            pltpu.SemaphoreType.DMA,
            pltpu.SemaphoreType.DMA,
            pltpu.SemaphoreType.DMA,
            pltpu.SemaphoreType.DMA,
            pltpu.SemaphoreType.DMA,
        ],
    )


def kernel(input_ids, embed_table, W, b):
    B, S = input_ids.shape
    P = _project_table(embed_table, W, b.reshape(1, HIDDEN))
    idx = input_ids.T.reshape(NW, NCH, C).astype(jnp.int32)
    out = _gather()(P, idx)
    return out.reshape(S, B, HIDDEN).transpose(1, 0, 2)


# speedup vs baseline: 1.0413x; 1.0057x over previous
import functools

import jax
import jax.numpy as jnp
from jax import lax
from jax.experimental import pallas as pl
from jax.experimental.pallas import tpu as pltpu
from jax.experimental.pallas import tpu_sc as plsc

VOCAB = 1000
HIDDEN = 256

NC = 2
NS = 16
NW = NC * NS

C = 128
NCH = 50
D = HIDDEN
HC = C // 2


def _proj_kernel(e_ref, w_ref, b_ref, out_ref):
    out_ref[...] = lax.dot_general(
        e_ref[...], w_ref[...],
        dimension_numbers=(((1,), (1,)), ((), ())),
        preferred_element_type=jnp.float32,
    ) + b_ref[...]


def _project_table(E, W, b2d):
    return pl.pallas_call(
        _proj_kernel,
        out_shape=jax.ShapeDtypeStruct((VOCAB, HIDDEN), jnp.float32),
    )(E, W, b2d)


def _gather_body(table_hbm, idx_hbm, out_hbm, shr, idx_v,
                 rows0, rows1, gsem0, gsem1, bsem0, bsem1, dsem0, dsem1):
    wid = lax.axis_index("s") * NC + lax.axis_index("c")
    sid = lax.axis_index("s")
    rows = (rows0, rows1)
    gsem = (gsem0, gsem1)
    bsem = (bsem0, bsem1)
    dsem = (dsem0, dsem1)
    spm = (shr.at[sid, 0], shr.at[sid, 1])
    row0 = wid * (NCH * C)
    pltpu.sync_copy(idx_hbm.at[wid], idx_v)

    g0 = pltpu.async_copy(table_hbm.at[idx_v.at[0]], rows[0], gsem0)
    g1 = pltpu.async_copy(table_hbm.at[idx_v.at[1]], rows[1], gsem1)
    del g0, g1

    @pl.loop(0, NCH, step=2)
    def _(ch):
        for b in range(2):
            cur = ch + b
            pltpu.make_async_copy(table_hbm.at[idx_v.at[cur]],
                                  rows[b], gsem[b]).wait()

            for h in range(2):
                @pl.when(cur * 2 + h >= 2)
                def _():
                    pltpu.make_async_copy(
                        spm[h],
                        out_hbm.at[pl.ds(row0 + cur * C + (h - 2) * HC, HC)],
                        dsem[h]).wait()

                half = rows[b].at[pl.ds(h * HC, HC)]
                pltpu.async_copy(half, spm[h], bsem[h])
                pltpu.make_async_copy(half, spm[h], bsem[h]).wait()
                pltpu.async_copy(
                    spm[h],
                    out_hbm.at[pl.ds(row0 + cur * C + h * HC, HC)],
                    dsem[h])

            @pl.when(cur + 2 < NCH)
            def _():
                pltpu.async_copy(table_hbm.at[idx_v.at[cur + 2]],
                                 rows[b], gsem[b])

    for h in range(2):
        pltpu.make_async_copy(
            spm[h],
            out_hbm.at[pl.ds(row0 + (NCH - 1) * C + h * HC, HC)],
            dsem[h]).wait()


@functools.cache
def _gather():
    return pl.kernel(
        _gather_body,
        out_type=jax.ShapeDtypeStruct((NW * NCH * C, D), jnp.float32),
        mesh=plsc.VectorSubcoreMesh(
            core_axis_name="c", subcore_axis_name="s",
            num_cores=NC, num_subcores=NS),
        compiler_params=pltpu.CompilerParams(use_tc_tiling_on_sc=True),
        scratch_types=[
            pltpu.VMEM_SHARED((NS, 2, HC, D), jnp.float32),
            pltpu.VMEM((NCH, C), jnp.int32),
            pltpu.VMEM((C, D), jnp.float32),
            pltpu.VMEM((C, D), jnp.float32),
            pltpu.SemaphoreType.DMA,
            pltpu.SemaphoreType.DMA,
            pltpu.SemaphoreType.DMA,
            pltpu.SemaphoreType.DMA,
            pltpu.SemaphoreType.DMA,
            pltpu.SemaphoreType.DMA,
        ],
    )


def kernel(input_ids, embed_table, W, b):
    B, S = input_ids.shape
    P = _project_table(embed_table, W, b.reshape(1, HIDDEN))
    idx = input_ids.T.reshape(NW, NCH, C).astype(jnp.int32)
    out = _gather()(P, idx)
    return out.reshape(S, B, HIDDEN).transpose(1, 0, 2)
